# baseline (speedup 1.0000x reference)
import functools

import jax
import jax.numpy as jnp
from jax import lax
from jax.experimental import pallas as pl
from jax.experimental.pallas import tpu as pltpu
from jax.experimental.pallas import tpu_sc as plsc

V = 100000
D = 64
DP = 128
L = 20
R_ALL = 71680
NW = 32
RPW = R_ALL // NW
CH = 224
CHP = 256
NCH = RPW // CH
NCHG = R_ALL // CH
HALF = 112
NG = CH // 16


def _sc_pool(idx_flat, len_pad, table):
    mesh = plsc.VectorSubcoreMesh(core_axis_name="c", subcore_axis_name="s")

    @functools.partial(
        pl.kernel,
        out_type=(
            jax.ShapeDtypeStruct((R_ALL * D,), jnp.float32),
            jax.ShapeDtypeStruct((NCHG * CHP,), jnp.float32),
        ),
        mesh=mesh,
        compiler_params=pltpu.CompilerParams(
            needs_layout_passes=False, use_tc_tiling_on_sc=True),
        scratch_types=[
            pltpu.VMEM((CH * L,), jnp.int32),
            pltpu.VMEM((CHP,), jnp.int32),
            pltpu.VMEM((2 * L * HALF,), jnp.int32),
            pltpu.VMEM((4 * HALF, DP), jnp.float32),
            pltpu.VMEM((CH * D,), jnp.float32),
            pltpu.VMEM((CHP,), jnp.float32),
            pltpu.SemaphoreType.DMA((4,)),
            pltpu.SemaphoreType.DMA,
        ],
    )
    def sc_kernel(idx_hbm, len_hbm, table_hbm, acc_hbm, nval_hbm,
                  idxr_v, len_v, idxt_v, rows_v, acc_v, nval_v, sem_g, sem_o):
        wid = lax.axis_index("s") * 2 + lax.axis_index("c")
        base = wid * RPW

        def wait_out():
            pltpu.make_async_copy(
                acc_v, acc_hbm.at[pl.ds(0, CH * D)], sem_o).wait()
            pltpu.make_async_copy(
                nval_v, nval_hbm.at[pl.ds(0, CHP)], sem_o).wait()

        def fire(j, b):
            pltpu.make_async_copy(
                table_hbm.at[idxt_v.at[pl.ds(j * HALF, HALF)]],
                rows_v.at[pl.ds(b * HALF, HALF), :],
                sem_g.at[b]).start()

        def wait_g(j, b):
            pltpu.make_async_copy(
                table_hbm.at[idxt_v.at[pl.ds(j * HALF, HALF)]],
                rows_v.at[pl.ds(b * HALF, HALF), :],
                sem_g.at[b]).wait()

        def accum(b, h):
            def arow(rr, _):
                rs = b * HALF + rr * 4
                rd = h * HALF + rr * 4
                for u in range(4):
                    for k in range(D // 16):
                        plsc.addupdate(
                            acc_v.at[pl.ds((rd + u) * D + k * 16, 16)],
                            rows_v[rs + u, pl.ds(k * 16, 16)])
                return _
            lax.fori_loop(0, HALF // 4, arow, None, unroll=False)

        def do_chunk(c, _):
            row0 = base + c * CH
            cg = row0 // CH
            pl.when(c > 0)(wait_out)
            pltpu.sync_copy(idx_hbm.at[pl.ds(row0 * L, CH * L)], idxr_v)
            pltpu.sync_copy(len_hbm.at[pl.ds(cg * CHP, CHP)], len_v)

            def grp(g, _):
                len_g = len_v[pl.ds(g * 16, 16)]
                h = g // 7
                col = (g - h * 7) * 16
                nv = jnp.zeros((16,), jnp.int32)
                for l in range(L):
                    iv = idxr_v[pl.ds(l * CH + g * 16, 16)]
                    valid = (l < len_g) & (iv != 0)
                    nv = nv + valid.astype(jnp.int32)
                    ivm = jnp.where(valid, iv, 0)
                    idxt_v[pl.ds((2 * l) * HALF + h * HALF + col, 16)] = ivm
                nval_v[pl.ds(g * 16, 16)] = nv.astype(jnp.float32)
                return _
            lax.fori_loop(0, NG, grp, None, unroll=False)

            for k in range(4):
                fire(k, k)

            def zrow(q, _):
                zz = jnp.zeros((16,), jnp.float32)
                for u in range(8):
                    acc_v[pl.ds(q * 128 + u * 16, 16)] = zz
                return _
            lax.fori_loop(0, CH * D // 128, zrow, None, unroll=False)

            def lstep(l2, _):
                j0 = 4 * l2
                for k in range(4):
                    wait_g(j0 + k, k)
                    accum(k, k % 2)
                    fire(j0 + k + 4, k)
                return _
            lax.fori_loop(0, (2 * L - 4) // 4, lstep, None, unroll=False)
            for k in range(4):
                wait_g(2 * L - 4 + k, k)
                accum(k, k % 2)

            pltpu.make_async_copy(
                acc_v, acc_hbm.at[pl.ds(row0 * D, CH * D)], sem_o).start()
            pltpu.make_async_copy(
                nval_v, nval_hbm.at[pl.ds(cg * CHP, CHP)], sem_o).start()
            return _

        lax.fori_loop(0, NCH, do_chunk, None, unroll=False)
        wait_out()

    return sc_kernel(idx_flat, len_pad, table)


BT = 256
GRID = 10240 // BT
BC = BT * 5


def _tc_epilogue(acc, nval, lens, t0, numc):
    def body(acc_t, nv_t, ln_t, acc_p, nv_p, ln_p, acc_c, nv_c, ln_c,
             t0_ref, num_ref, out_t, out_p, out_c):
        t0v = t0_ref[...]

        def mean(a, nv, ln):
            corr = (jnp.float32(L) - nv) * t0v
            return jnp.where(ln > 0.0, (a - corr) / ln, 0.0)

        out_t[...] = mean(acc_t[...], nv_t[...], ln_t[...])
        out_p[...] = mean(acc_p[...], nv_p[...], ln_p[...])
        m3 = mean(acc_c[...], nv_c[...], ln_c[...]).reshape(BT, 5, D)
        numv = num_ref[...]
        nmask = (lax.broadcasted_iota(jnp.int32, (BT, 5, 1), 1).astype(
            jnp.float32) < numv[:, :, None])
        s = jnp.sum(jnp.where(nmask, m3, 0.0), axis=1)
        out_c[...] = jnp.where(numv > 0.0, s / numv, 0.0)

    def rows(n):
        return pl.BlockSpec((n, D), lambda i, n=n: (i, 0))

    def rows_off(n, off):
        return pl.BlockSpec((n, D), lambda i, off=off, n=n: (i + off, 0))

    def col(n):
        return pl.BlockSpec((n, 1), lambda i, n=n: (i, 0))

    def col_off(n, off):
        return pl.BlockSpec((n, 1), lambda i, off=off, n=n: (i + off, 0))

    return pl.pallas_call(
        body,
        grid=(GRID,),
        in_specs=[
            rows(BT), col(BT), col(BT),
            rows_off(BT, GRID), col_off(BT, GRID), col_off(BT, GRID),
            rows_off(BC, 2 * GRID // 5), col_off(BC, 2 * GRID // 5),
            col_off(BC, 2 * GRID // 5),
            pl.BlockSpec((1, D), lambda i: (0, 0)),
            col(BT),
        ],
        out_specs=[rows(BT), rows(BT), rows(BT)],
        out_shape=[jax.ShapeDtypeStruct((10240, D), jnp.float32)] * 3,
    )(acc, nval, lens, acc, nval, lens, acc, nval, lens, t0, numc)


def kernel(x_type_bow, x_types, x_type_bow_len, x_path_bow, x_paths,
           x_path_bow_len, x_ctx_ents, x_ctx_ent_len, x_ctx_ent_num,
           embed_weight):
    B, C, _ = x_type_bow.shape
    idx_all = jnp.concatenate([
        x_type_bow.reshape(-1, L),
        x_path_bow.reshape(-1, L),
        x_ctx_ents.reshape(-1, L),
    ], axis=0)
    idx_slab = idx_all.reshape(-1, CH, L).transpose(0, 2, 1)
    len_all = jnp.concatenate([
        x_type_bow_len.reshape(-1),
        x_path_bow_len.reshape(-1),
        x_ctx_ent_len.reshape(-1),
    ])
    len_pad = jnp.pad(len_all.reshape(NCHG, CH), ((0, 0), (0, CHP - CH)))
    table_pad = jnp.concatenate(
        [embed_weight, jnp.zeros_like(embed_weight)], axis=1)
    acc, nval = _sc_pool(idx_slab.reshape(-1), len_pad.reshape(-1), table_pad)
    nval = nval.reshape(NCHG, CHP)[:, :CH].reshape(-1, 1)
    lens_f = len_all.astype(jnp.float32).reshape(-1, 1)
    out_t, out_p, out_c = _tc_epilogue(
        acc.reshape(R_ALL, D), nval, lens_f, embed_weight[0:1, :],
        x_ctx_ent_num.reshape(-1, 1).astype(jnp.float32))
    return (out_t.reshape(B, C, D), out_p.reshape(B, C, D),
            out_c.reshape(B, C, D))


# speedup vs baseline: 3.5317x; 3.5317x over previous
import functools

import jax
import jax.numpy as jnp
from jax import lax
from jax.experimental import pallas as pl
from jax.experimental.pallas import tpu as pltpu
from jax.experimental.pallas import tpu_sc as plsc

V = 100000
D = 64
DP = 128
L = 20
R_ALL = 71680
NW = 32
RPW = R_ALL // NW
CH = 224
CHP = 256
NCH = RPW // CH
NCHG = R_ALL // CH
HALF = 112
NG = CH // 16


def _sc_pool(idx_flat, len_pad, table):
    mesh = plsc.VectorSubcoreMesh(core_axis_name="c", subcore_axis_name="s")

    @functools.partial(
        pl.kernel,
        out_type=(
            jax.ShapeDtypeStruct((R_ALL * D,), jnp.float32),
            jax.ShapeDtypeStruct((NCHG * CHP,), jnp.float32),
        ),
        mesh=mesh,
        compiler_params=pltpu.CompilerParams(
            needs_layout_passes=False, use_tc_tiling_on_sc=False),
        scratch_types=[
            pltpu.VMEM((CH * L,), jnp.int32),
            pltpu.VMEM((CHP,), jnp.int32),
            pltpu.VMEM((2 * L * HALF,), jnp.int32),
            pltpu.VMEM((4 * HALF, D // 2), jnp.int32),
            pltpu.VMEM((CH * D,), jnp.float32),
            pltpu.VMEM((CHP,), jnp.float32),
            pltpu.SemaphoreType.DMA((4,)),
            pltpu.SemaphoreType.DMA,
        ],
    )
    def sc_kernel(idx_hbm, len_hbm, table_hbm, acc_hbm, nval_hbm,
                  idxr_v, len_v, idxt_v, rows_v, acc_v, nval_v, sem_g, sem_o):
        wid = lax.axis_index("s") * 2 + lax.axis_index("c")
        base = wid * RPW

        def wait_out():
            pltpu.make_async_copy(
                acc_v, acc_hbm.at[pl.ds(0, CH * D)], sem_o).wait()
            pltpu.make_async_copy(
                nval_v, nval_hbm.at[pl.ds(0, CHP)], sem_o).wait()

        def fire(j, b):
            pltpu.make_async_copy(
                table_hbm.at[idxt_v.at[pl.ds(j * HALF, HALF)]],
                rows_v.at[pl.ds(b * HALF, HALF), :],
                sem_g.at[b]).start()

        def wait_g(j, b):
            pltpu.make_async_copy(
                table_hbm.at[idxt_v.at[pl.ds(j * HALF, HALF)]],
                rows_v.at[pl.ds(b * HALF, HALF), :],
                sem_g.at[b]).wait()

        def accum(b, h):
            def arow(rr, _):
                rs = b * HALF + rr * 4
                rd = h * HALF + rr * 4
                for u in range(4):
                    for k in range(D // 32):
                        w = rows_v[rs + u, pl.ds(k * 16, 16)]
                        ab = plsc.bitcast(w, jnp.bfloat16)
                        ea, eb = plsc.unpack(
                            ab, format=plsc.PackFormat.INTERLEAVED)
                        base = (rd + u) * D + k * 32
                        plsc.addupdate(acc_v.at[pl.ds(base, 16)], ea)
                        plsc.addupdate(acc_v.at[pl.ds(base + 16, 16)], eb)
                return _
            lax.fori_loop(0, HALF // 4, arow, None, unroll=False)

        def do_chunk(c, _):
            row0 = base + c * CH
            cg = row0 // CH
            pl.when(c > 0)(wait_out)
            pltpu.sync_copy(idx_hbm.at[pl.ds(row0 * L, CH * L)], idxr_v)
            pltpu.sync_copy(len_hbm.at[pl.ds(cg * CHP, CHP)], len_v)

            def grp(g, _):
                len_g = len_v[pl.ds(g * 16, 16)]
                h = g // 7
                col = (g - h * 7) * 16
                nv = jnp.zeros((16,), jnp.int32)
                for l in range(L):
                    iv = idxr_v[pl.ds(l * CH + g * 16, 16)]
                    valid = (l < len_g) & (iv != 0)
                    nv = nv + valid.astype(jnp.int32)
                    ivm = jnp.where(valid, iv, 0)
                    idxt_v[pl.ds((2 * l) * HALF + h * HALF + col, 16)] = ivm
                nval_v[pl.ds(g * 16, 16)] = nv.astype(jnp.float32)
                return _
            lax.fori_loop(0, NG, grp, None, unroll=False)

            for k in range(4):
                fire(k, k)

            def zrow(q, _):
                zz = jnp.zeros((16,), jnp.float32)
                for u in range(8):
                    acc_v[pl.ds(q * 128 + u * 16, 16)] = zz
                return _
            lax.fori_loop(0, CH * D // 128, zrow, None, unroll=False)

            def lstep(l2, _):
                j0 = 4 * l2
                for k in range(4):
                    wait_g(j0 + k, k)
                    accum(k, k % 2)
                    fire(j0 + k + 4, k)
                return _
            lax.fori_loop(0, (2 * L - 4) // 4, lstep, None, unroll=False)
            for k in range(4):
                wait_g(2 * L - 4 + k, k)
                accum(k, k % 2)

            pltpu.make_async_copy(
                acc_v, acc_hbm.at[pl.ds(row0 * D, CH * D)], sem_o).start()
            pltpu.make_async_copy(
                nval_v, nval_hbm.at[pl.ds(cg * CHP, CHP)], sem_o).start()
            return _

        lax.fori_loop(0, NCH, do_chunk, None, unroll=False)
        wait_out()

    return sc_kernel(idx_flat, len_pad, table)


BT = 256
GRID = 10240 // BT
BC = BT * 5


def _tc_epilogue(acc, nval, lens, t0, numc):
    def body(acc_t, nv_t, ln_t, acc_p, nv_p, ln_p, acc_c, nv_c, ln_c,
             t0_ref, num_ref, out_t, out_p, out_c):
        t0v = t0_ref[...]

        def mean(a, nv, ln):
            corr = (jnp.float32(L) - nv) * t0v
            return jnp.where(ln > 0.0, (a - corr) / ln, 0.0)

        out_t[...] = mean(acc_t[...], nv_t[...], ln_t[...])
        out_p[...] = mean(acc_p[...], nv_p[...], ln_p[...])
        m3 = mean(acc_c[...], nv_c[...], ln_c[...]).reshape(BT, 5, D)
        numv = num_ref[...]
        nmask = (lax.broadcasted_iota(jnp.int32, (BT, 5, 1), 1).astype(
            jnp.float32) < numv[:, :, None])
        s = jnp.sum(jnp.where(nmask, m3, 0.0), axis=1)
        out_c[...] = jnp.where(numv > 0.0, s / numv, 0.0)

    def rows(n):
        return pl.BlockSpec((n, D), lambda i, n=n: (i, 0))

    def rows_off(n, off):
        return pl.BlockSpec((n, D), lambda i, off=off, n=n: (i + off, 0))

    def col(n):
        return pl.BlockSpec((n, 1), lambda i, n=n: (i, 0))

    def col_off(n, off):
        return pl.BlockSpec((n, 1), lambda i, off=off, n=n: (i + off, 0))

    return pl.pallas_call(
        body,
        grid=(GRID,),
        in_specs=[
            rows(BT), col(BT), col(BT),
            rows_off(BT, GRID), col_off(BT, GRID), col_off(BT, GRID),
            rows_off(BC, 2 * GRID // 5), col_off(BC, 2 * GRID // 5),
            col_off(BC, 2 * GRID // 5),
            pl.BlockSpec((1, D), lambda i: (0, 0)),
            col(BT),
        ],
        out_specs=[rows(BT), rows(BT), rows(BT)],
        out_shape=[jax.ShapeDtypeStruct((10240, D), jnp.float32)] * 3,
    )(acc, nval, lens, acc, nval, lens, acc, nval, lens, t0, numc)


def kernel(x_type_bow, x_types, x_type_bow_len, x_path_bow, x_paths,
           x_path_bow_len, x_ctx_ents, x_ctx_ent_len, x_ctx_ent_num,
           embed_weight):
    B, C, _ = x_type_bow.shape
    idx_all = jnp.concatenate([
        x_type_bow.reshape(-1, L),
        x_path_bow.reshape(-1, L),
        x_ctx_ents.reshape(-1, L),
    ], axis=0)
    idx_slab = idx_all.reshape(-1, CH, L).transpose(0, 2, 1)
    len_all = jnp.concatenate([
        x_type_bow_len.reshape(-1),
        x_path_bow_len.reshape(-1),
        x_ctx_ent_len.reshape(-1),
    ])
    len_pad = jnp.pad(len_all.reshape(NCHG, CH), ((0, 0), (0, CHP - CH)))
    table_bf = embed_weight.astype(jnp.bfloat16)
    table32 = lax.bitcast_convert_type(
        table_bf.reshape(V, D // 2, 2), jnp.int32)
    acc, nval = _sc_pool(idx_slab.reshape(-1), len_pad.reshape(-1), table32)
    acc = (acc.reshape(R_ALL, D // 32, 2, 16).transpose(0, 1, 3, 2)
           .reshape(R_ALL, D))
    nval = nval.reshape(NCHG, CHP)[:, :CH].reshape(-1, 1)
    lens_f = len_all.astype(jnp.float32).reshape(-1, 1)
    t0 = embed_weight[0:1, :].astype(jnp.bfloat16).astype(jnp.float32)
    out_t, out_p, out_c = _tc_epilogue(
        acc, nval, lens_f, t0,
        x_ctx_ent_num.reshape(-1, 1).astype(jnp.float32))
    return (out_t.reshape(B, C, D), out_p.reshape(B, C, D),
            out_c.reshape(B, C, D))
